# Initial kernel scaffold; baseline (speedup 1.0000x reference)
import jax
import jax.numpy as jnp
from jax.experimental import pallas as pl


def kernel(hidden_states, W_in, b_in, embed, W_out, b_out):
    raise NotImplementedError("write your pallas kernel here")


# speedup vs baseline: 1.6697x; 1.6697x over previous
import jax
import jax.numpy as jnp
from jax.experimental import pallas as pl

_HI = jax.lax.Precision.HIGHEST


def _vq_body(x_ref, w_in_ref, b_in_ref, emb_ref, w_out_ref, b_out_ref, o_ref):
    X = x_ref[0]
    Wi = w_in_ref[...]
    E = emb_ref[...]
    Wo = w_out_ref[...]

    Z = jax.lax.dot_general(X, Wi, (((0,), (0,)), ((), ())),
                            preferred_element_type=jnp.float32)
    Z = Z + b_in_ref[...]

    x2 = jnp.sum(Z * Z, axis=1, keepdims=True)
    S = jax.lax.dot_general(Z, E, (((1,), (1,)), ((), ())),
                            preferred_element_type=jnp.float32)
    ones = jnp.ones((1, E.shape[1]), jnp.float32)
    e2 = jax.lax.dot_general(ones, E * E, (((1,), (1,)), ((), ())),
                             precision=_HI, preferred_element_type=jnp.float32)
    dist = -(x2 - 2.0 * S + e2)

    ind = jnp.argmax(dist, axis=1)
    iota = jax.lax.broadcasted_iota(jnp.int32, dist.shape, 1)
    oh = (iota == ind[:, None]).astype(jnp.float32)

    Q = jax.lax.dot_general(E, oh, (((0,), (1,)), ((), ())),
                            preferred_element_type=jnp.float32)
    Y = jax.lax.dot_general(Wo, Q, (((0,), (0,)), ((), ())),
                            preferred_element_type=jnp.float32)
    o_ref[0] = Y + b_out_ref[...]


def kernel(hidden_states, W_in, b_in, embed, W_out, b_out):
    B, D, T = hidden_states.shape
    K, CD = embed.shape
    Tb = 512

    b_in2 = b_in.reshape(1, CD)
    b_out2 = b_out.reshape(D, 1)

    grid = (B, T // Tb)
    out = pl.pallas_call(
        _vq_body,
        grid=grid,
        in_specs=[
            pl.BlockSpec((1, D, Tb), lambda b, t: (b, 0, t)),
            pl.BlockSpec((D, CD), lambda b, t: (0, 0)),
            pl.BlockSpec((1, CD), lambda b, t: (0, 0)),
            pl.BlockSpec((K, CD), lambda b, t: (0, 0)),
            pl.BlockSpec((CD, D), lambda b, t: (0, 0)),
            pl.BlockSpec((D, 1), lambda b, t: (0, 0)),
        ],
        out_specs=pl.BlockSpec((1, D, Tb), lambda b, t: (b, 0, t)),
        out_shape=jax.ShapeDtypeStruct((B, D, T), jnp.float32),
    )(hidden_states, W_in, b_in2, embed, W_out, b_out2)
    return out


# speedup vs baseline: 2.5926x; 1.5527x over previous
import jax
import jax.numpy as jnp
from jax.experimental import pallas as pl

_HI = jax.lax.Precision.HIGHEST


def _e2_body(emb_ref, o_ref):
    E = emb_ref[...]
    ones = jnp.ones((8, E.shape[1]), jnp.float32)
    o_ref[...] = jax.lax.dot_general(
        ones, E * E, (((1,), (1,)), ((), ())),
        precision=_HI, preferred_element_type=jnp.float32)


def _vq_body(x_ref, w_in_ref, b_in_ref, emb_ref, w_out_ref, b_out_ref,
             e2_ref, o_ref):
    X = x_ref[0]
    Wi = w_in_ref[...]
    E = emb_ref[...]
    Wo = w_out_ref[...]

    Z = jax.lax.dot_general(X, Wi, (((0,), (0,)), ((), ())),
                            preferred_element_type=jnp.float32)
    Z = Z + b_in_ref[...]

    x2 = jnp.sum(Z * Z, axis=1, keepdims=True)
    S = jax.lax.dot_general(Z, E, (((1,), (1,)), ((), ())),
                            preferred_element_type=jnp.float32)
    dist = -(x2 - 2.0 * S + e2_ref[...])

    ind = jnp.argmax(dist, axis=1)
    iota = jax.lax.broadcasted_iota(jnp.int32, dist.shape, 1)
    oh = (iota == ind[:, None]).astype(jnp.float32)

    Q = jax.lax.dot_general(E, oh, (((0,), (1,)), ((), ())),
                            preferred_element_type=jnp.float32)
    Y = jax.lax.dot_general(Wo, Q, (((0,), (0,)), ((), ())),
                            preferred_element_type=jnp.float32)
    o_ref[0] = Y + b_out_ref[...]


def kernel(hidden_states, W_in, b_in, embed, W_out, b_out):
    B, D, T = hidden_states.shape
    K, CD = embed.shape
    Tb = 1024

    b_in2 = b_in.reshape(1, CD)
    b_out2 = b_out.reshape(D, 1)

    e2 = pl.pallas_call(
        _e2_body,
        out_shape=jax.ShapeDtypeStruct((8, K), jnp.float32),
    )(embed)[0:1]

    grid = (B, T // Tb)
    out = pl.pallas_call(
        _vq_body,
        grid=grid,
        in_specs=[
            pl.BlockSpec((1, D, Tb), lambda b, t: (b, 0, t)),
            pl.BlockSpec((D, CD), lambda b, t: (0, 0)),
            pl.BlockSpec((1, CD), lambda b, t: (0, 0)),
            pl.BlockSpec((K, CD), lambda b, t: (0, 0)),
            pl.BlockSpec((CD, D), lambda b, t: (0, 0)),
            pl.BlockSpec((D, 1), lambda b, t: (0, 0)),
            pl.BlockSpec((1, K), lambda b, t: (0, 0)),
        ],
        out_specs=pl.BlockSpec((1, D, Tb), lambda b, t: (b, 0, t)),
        out_shape=jax.ShapeDtypeStruct((B, D, T), jnp.float32),
    )(hidden_states, W_in, b_in2, embed, W_out, b_out2, e2)
    return out
